# baseline (speedup 1.0000x reference)
import functools

import jax
import jax.numpy as jnp
from jax import lax
from jax.experimental import pallas as pl
from jax.experimental.pallas import tpu as pltpu
from jax.experimental.pallas import tpu_sc as plsc

NP_ = 576
PD_ = 768
B_ = 256

NC_ = 2
NS_ = 16
NW_ = NC_ * NS_
BPW_ = B_ // NW_
PC_ = 16
NPC_ = NP_ // PC_
NCHUNK_ = NPC_ * BPW_
NBUF_ = 8
VPR_ = PD_ // 16
NSTEP_ = NPC_ // 2

_mesh = plsc.VectorSubcoreMesh(core_axis_name="c", subcore_axis_name="s")


@functools.partial(
    pl.kernel,
    out_type=jax.ShapeDtypeStruct((B_, NP_, PD_), jnp.float32),
    mesh=_mesh,
    scratch_types=(
        [pltpu.VMEM((PC_, PD_), jnp.float32) for _ in range(2)]
        + [pltpu.VMEM((PC_, PD_), jnp.float32) for _ in range(NBUF_)]
        + [pltpu.SemaphoreType.DMA for _ in range(2 + 2 * NBUF_)]
    ),
)
def _sc_add(x_hbm, t_hbm, out_hbm, *scratch):
    t_bufs = scratch[:2]
    bufs = scratch[2:2 + NBUF_]
    t_sems = scratch[2 + NBUF_:4 + NBUF_]
    in_sems = scratch[4 + NBUF_:4 + 2 * NBUF_]
    out_sems = scratch[4 + 2 * NBUF_:]

    w = lax.axis_index("s") * NC_ + lax.axis_index("c")
    b0 = w * BPW_

    def x_slice(ref, pc, v):
        return ref.at[b0 + (v % NBUF_), pl.ds(pc * PC_, PC_)]

    def t_slice(pc):
        return t_hbm.at[pl.ds(pc * PC_, PC_)]

    pltpu.sync_copy(t_slice(0), t_bufs[0])
    for v in range(NBUF_ // 2):
        pltpu.async_copy(x_slice(x_hbm, 0, v), bufs[v], in_sems[v])

    def step(j, carry):
        for v in range(2 * NBUF_):
            pc = 2 * j + v // NBUF_
            ph = v % NBUF_
            bph = (ph + NBUF_ // 2) % NBUF_
            t_buf = t_bufs[v // NBUF_]

            if v == 0:
                def wait_t0():
                    pltpu.make_async_copy(t_slice(2 * j), t_bufs[0], t_sems[0]).wait()

                pl.when(j >= 1)(wait_t0)
                pltpu.async_copy(t_slice(2 * j + 1), t_bufs[1], t_sems[1])
            elif v == NBUF_:
                pltpu.make_async_copy(t_slice(2 * j + 1), t_bufs[1], t_sems[1]).wait()

                def prefetch_t0():
                    pltpu.async_copy(t_slice(2 * j + 2), t_bufs[0], t_sems[0])

                pl.when(j < NSTEP_ - 1)(prefetch_t0)

            pcp = 2 * j + (v - NBUF_ // 2) // NBUF_
            pcn = 2 * j + (v + NBUF_ // 2) // NBUF_

            def drain_buddy():
                pltpu.make_async_copy(
                    bufs[bph], x_slice(out_hbm, pcp, v - NBUF_ // 2), out_sems[bph]
                ).wait()

            def prefetch_buddy():
                pltpu.async_copy(
                    x_slice(x_hbm, pcn, v + NBUF_ // 2), bufs[bph], in_sems[bph]
                )

            if v < NBUF_ // 2:
                pl.when(j >= 1)(drain_buddy)
                prefetch_buddy()
            elif v < 2 * NBUF_ - NBUF_ // 2:
                drain_buddy()
                prefetch_buddy()
            else:
                drain_buddy()
                pl.when(j < NSTEP_ - 1)(prefetch_buddy)

            pltpu.make_async_copy(x_slice(x_hbm, pc, v), bufs[ph], in_sems[ph]).wait()

            buf = bufs[ph]

            @plsc.parallel_loop(0, PC_, unroll=2)
            def add_body(r):
                for c in range(VPR_):
                    plsc.addupdate(
                        buf.at[r, pl.ds(c * 16, 16)], t_buf[r, pl.ds(c * 16, 16)]
                    )

            pltpu.async_copy(buf, x_slice(out_hbm, pc, v), out_sems[ph])
        return carry

    lax.fori_loop(0, NSTEP_, step, 0)

    for v in range(2 * NBUF_ - NBUF_ // 2, 2 * NBUF_):
        pltpu.make_async_copy(
            bufs[v % NBUF_], x_slice(out_hbm, NPC_ - 1, v), out_sems[v % NBUF_]
        ).wait()


def kernel(encoded_patches, pos_table):
    return _sc_add(encoded_patches, pos_table)
